# baseline (speedup 1.0000x reference)
import functools

import jax
import jax.numpy as jnp
from jax import lax
from jax.experimental import pallas as pl
from jax.experimental.pallas import tpu as pltpu
from jax.experimental.pallas import tpu_sc as plsc

N = 10000
E = 320000
D = 128
Q = 32
NUM_REL = 2
HALF = N // 2
P = HALF + 120
LPR = 128
EROWS = 2560
EPAD = EROWS * LPR
RPT = EROWS // 16
SUP = 32
OPT = P // 16
ZPT = 2 * P // 16


def _lrelu(x):
    return jnp.where(x > 0, x, 0.01 * x)


def _stage_a_body(des, tweet, prop, wd, bd, wt, bt, wp, bp, wi, bi,
                  rw, root, rbias, h_out, r_out):
    d = _lrelu(jnp.dot(des[...], wd[...], preferred_element_type=jnp.float32) + bd[...])
    t = _lrelu(jnp.dot(tweet[...], wt[...], preferred_element_type=jnp.float32) + bt[...])
    p = _lrelu(jnp.dot(prop[...], wp[...], preferred_element_type=jnp.float32) + bp[...])
    x = jnp.concatenate((d, t, p), axis=1)
    x = _lrelu(jnp.dot(x, wi[...], preferred_element_type=jnp.float32) + bi[...])
    h_out[0] = jnp.dot(x, rw[0], preferred_element_type=jnp.float32)
    h_out[1] = jnp.dot(x, rw[1], preferred_element_type=jnp.float32)
    r_out[...] = jnp.dot(x, root[...], preferred_element_type=jnp.float32) + rbias[...]


def _stage_c_body(r0, agg, cnt, rw, root, rbias, h_out, r_out):
    c0 = jnp.maximum(cnt[0][:, 0:1], 1.0)
    c1 = jnp.maximum(cnt[1][:, 0:1], 1.0)
    x = r0[...] + agg[0] / c0 + agg[1] / c1
    h_out[0] = jnp.dot(x, rw[0], preferred_element_type=jnp.float32)
    h_out[1] = jnp.dot(x, rw[1], preferred_element_type=jnp.float32)
    r_out[...] = jnp.dot(x, root[...], preferred_element_type=jnp.float32) + rbias[...]


def _stage_e_body(r1, agg, cnt, wo1, bo1, wo2, bo2, out):
    c0 = jnp.maximum(cnt[0][:, 0:1], 1.0)
    c1 = jnp.maximum(cnt[1][:, 0:1], 1.0)
    x = r1[...] + agg[0] / c0 + agg[1] / c1
    h = _lrelu(jnp.dot(x, wo1[...], preferred_element_type=jnp.float32) + bo1[...])
    lg = jnp.dot(h, wo2[...], preferred_element_type=jnp.float32) + bo2[...]
    m = jnp.max(lg, axis=-1, keepdims=True)
    e = jnp.exp(lg - m)
    out[...] = e / jnp.sum(e, axis=-1, keepdims=True)


def _scatter_idx(dstb, typb, r, base, trash, sall):
    for l in range(8):
        sl = pl.ds(l * 16, 16)
        dv = dstb[r, sl]
        tv = typb[r, sl]
        rel = tv * P + (dv - base)
        ok = (dv >= base) & (dv < base + HALF)
        sall[r, sl] = jnp.where(ok, rel, trash)


GRO = 32
EROWS2 = EPAD // GRO
RPT2 = EROWS2 // 16
SUP2 = 64
NBUF = 4


def _conv_body(h, srcr, dstr, typr, z128,
               a_out, acc, dstb, gall, sall,
               rows0, rows1, rows2, rows3, sem0, sem1, sem2, sem3):
    c = lax.axis_index("c")
    s = lax.axis_index("s")
    lane = lax.iota(jnp.int32, 16)
    trash = HALF + (lane & 7) + P * (lane >> 3)
    base = c * HALF
    bufs = (rows0, rows1, rows2, rows3)
    sems = (sem0, sem1, sem2, sem3)

    pltpu.sync_copy(z128, rows0)
    for k in range(ZPT // GRO):
        pltpu.sync_copy(rows0, acc.at[pl.ds(s * ZPT + k * GRO, GRO)])
    plsc.subcore_barrier()

    def idx_step(r, _):
        for l in range(GRO // 16):
            sl = pl.ds(l * 16, 16)
            sv = gall[r, sl]
            tv = sall[r, sl]
            dv = dstb[r, sl]
            rel = tv * P + (dv - base)
            ok = (dv >= base) & (dv < base + HALF)
            gall[r, sl] = tv * N + sv
            sall[r, sl] = jnp.where(ok, rel, trash)
        return _

    def quad_step(k, _):
        for j in range(NBUF):
            r = NBUF * k + j
            pltpu.make_async_copy(h.at[gall.at[r]], bufs[j], sems[j]).wait()
            pltpu.sync_copy(bufs[j], acc.at[sall.at[r]], add=True)

            @pl.when(r + NBUF < SUP2)
            def _prefetch():
                pltpu.async_copy(h.at[gall.at[r + NBUF]], bufs[j], sems[j])
        return _

    for sc_i in range(RPT2 // SUP2):
        j0 = s * RPT2 + sc_i * SUP2
        pltpu.sync_copy(srcr.at[pl.ds(j0, SUP2)], gall)
        pltpu.sync_copy(dstr.at[pl.ds(j0, SUP2)], dstb)
        pltpu.sync_copy(typr.at[pl.ds(j0, SUP2)], sall)
        lax.fori_loop(0, SUP2, idx_step, None)
        for j in range(NBUF):
            pltpu.async_copy(h.at[gall.at[j]], bufs[j], sems[j])
        lax.fori_loop(0, SUP2 // NBUF, quad_step, None)

    plsc.subcore_barrier()

    for t in range(2):
        pltpu.sync_copy(acc.at[pl.ds(t * P + s * OPT, OPT)],
                        a_out.at[c, t, pl.ds(s * OPT, OPT)])


def _count_body(dstr, typr, z128, o128,
                c_out, cacc, dstb, sall, obuf):
    c = lax.axis_index("c")
    s = lax.axis_index("s")
    lane = lax.iota(jnp.int32, 16)
    trash = HALF + (lane & 7) + P * (lane >> 3)
    base = c * HALF

    pltpu.sync_copy(z128, obuf)
    for k in range(ZPT // 128):
        pltpu.sync_copy(obuf, cacc.at[pl.ds(s * ZPT + k * 128, 128)])
    pltpu.sync_copy(o128, obuf)
    plsc.subcore_barrier()

    def row_step(r, _):
        for l in range(8):
            sl = pl.ds(l * 16, 16)
            dv = dstb[r, sl]
            tv = sall[r, sl]
            rel = tv * P + (dv - base)
            ok = (dv >= base) & (dv < base + HALF)
            sall[r, sl] = jnp.where(ok, rel, trash)
        pltpu.sync_copy(obuf, cacc.at[sall.at[r]], add=True)
        return _

    for sc_i in range(RPT // SUP):
        j0 = s * RPT + sc_i * SUP
        pltpu.sync_copy(dstr.at[pl.ds(j0, SUP)], dstb)
        pltpu.sync_copy(typr.at[pl.ds(j0, SUP)], sall)
        lax.fori_loop(0, SUP, row_step, None)

    plsc.subcore_barrier()
    for t in range(2):
        pltpu.sync_copy(cacc.at[pl.ds(t * P + s * OPT, OPT)],
                        c_out.at[c, t, pl.ds(s * OPT, OPT)])


def _sc_mesh():
    return plsc.VectorSubcoreMesh(core_axis_name="c", subcore_axis_name="s",
                                  num_cores=2, num_subcores=16)


def _make_conv(interpret=False):
    f32 = jnp.float32
    return pl.kernel(
        _conv_body,
        out_type=jax.ShapeDtypeStruct((2, 2, P, 128), f32),
        mesh=_sc_mesh(),
        scratch_types=[
            pltpu.VMEM_SHARED((2 * P, 128), f32),
            pltpu.VMEM((SUP2, GRO), jnp.int32),
            pltpu.VMEM((SUP2, GRO), jnp.int32),
            pltpu.VMEM((SUP2, GRO), jnp.int32),
            pltpu.VMEM((GRO, 128), f32),
            pltpu.VMEM((GRO, 128), f32),
            pltpu.VMEM((GRO, 128), f32),
            pltpu.VMEM((GRO, 128), f32),
            pltpu.SemaphoreType.DMA,
            pltpu.SemaphoreType.DMA,
            pltpu.SemaphoreType.DMA,
            pltpu.SemaphoreType.DMA,
        ],
        interpret=interpret,
    )


def _make_count(interpret=False):
    f32 = jnp.float32
    return pl.kernel(
        _count_body,
        out_type=jax.ShapeDtypeStruct((2, 2, P, 128), f32),
        mesh=_sc_mesh(),
        scratch_types=[
            pltpu.VMEM_SHARED((2 * P, 128), f32),
            pltpu.VMEM((SUP, 128), jnp.int32),
            pltpu.VMEM((SUP, 128), jnp.int32),
            pltpu.VMEM((128, 128), f32),
        ],
        interpret=interpret,
    )


def kernel(des, tweet, prop, edge_index, edge_type,
           W_des, b_des, W_tweet, b_tweet, W_prop, b_prop,
           W_in, b_in, rgcn_weight, rgcn_root, rgcn_bias,
           W_o1, b_o1, W_o2, b_o2):
    f32 = jnp.float32
    B = 400
    grid = (N // B,)

    full = lambda shp: pl.BlockSpec(shp, lambda i: (0,) * len(shp))
    row2 = lambda shp: pl.BlockSpec(shp, lambda i: (i, 0))

    bd = b_des.reshape(1, Q)
    bt = b_tweet.reshape(1, Q)
    bp = b_prop.reshape(1, Q)
    bi = b_in.reshape(1, D)
    rb = rgcn_bias.reshape(1, D)
    bo1 = b_o1.reshape(1, D)
    bo2 = b_o2.reshape(1, 2)

    h0, r0 = pl.pallas_call(
        _stage_a_body,
        grid=grid,
        in_specs=[
            row2((B, 768)), row2((B, 768)), row2((B, 14)),
            full((768, Q)), full((1, Q)),
            full((768, Q)), full((1, Q)),
            full((14, Q)), full((1, Q)),
            full((3 * Q, D)), full((1, D)),
            full((NUM_REL, D, D)), full((D, D)), full((1, D)),
        ],
        out_specs=[
            pl.BlockSpec((NUM_REL, B, D), lambda i: (0, i, 0)),
            row2((B, D)),
        ],
        out_shape=[
            jax.ShapeDtypeStruct((NUM_REL, N, D), f32),
            jax.ShapeDtypeStruct((N, D), f32),
        ],
    )(des, tweet, prop, W_des, bd, W_tweet, bt, W_prop, bp, W_in, bi,
      rgcn_weight, rgcn_root, rb)

    pad = EPAD - E
    src = jnp.concatenate(
        [edge_index[0], (jnp.arange(pad, dtype=jnp.int32) * 97) % N]
    ).reshape(EROWS, LPR)
    dst = jnp.concatenate(
        [edge_index[1], jnp.full((pad,), -1, jnp.int32)]).reshape(EROWS, LPR)
    typ = jnp.concatenate(
        [edge_type, jnp.zeros((pad,), jnp.int32)]).reshape(EROWS, LPR)
    z128 = jnp.zeros((128, 128), f32)
    o128 = jnp.ones((128, 128), f32)
    z64 = jnp.zeros((GRO, 128), f32)
    src2 = src.reshape(EROWS2, GRO)
    dst2 = dst.reshape(EROWS2, GRO)
    typ2 = typ.reshape(EROWS2, GRO)

    cnt = _make_count()(dst, typ, z128, o128)
    cntf = jnp.concatenate([cnt[0, :, :HALF], cnt[1, :, :HALF]], axis=1)

    a0 = _make_conv()(h0.reshape(NUM_REL * N, D), src2, dst2, typ2, z64)
    aggr0 = jnp.concatenate([a0[0, :, :HALF], a0[1, :, :HALF]], axis=1)

    h1, r1 = pl.pallas_call(
        _stage_c_body,
        grid=grid,
        in_specs=[
            row2((B, D)),
            pl.BlockSpec((NUM_REL, B, D), lambda i: (0, i, 0)),
            pl.BlockSpec((NUM_REL, B, D), lambda i: (0, i, 0)),
            full((NUM_REL, D, D)), full((D, D)), full((1, D)),
        ],
        out_specs=[
            pl.BlockSpec((NUM_REL, B, D), lambda i: (0, i, 0)),
            row2((B, D)),
        ],
        out_shape=[
            jax.ShapeDtypeStruct((NUM_REL, N, D), f32),
            jax.ShapeDtypeStruct((N, D), f32),
        ],
    )(r0, aggr0, cntf, rgcn_weight, rgcn_root, rb)

    a1 = _make_conv()(h1.reshape(NUM_REL * N, D), src2, dst2, typ2, z64)
    aggr1 = jnp.concatenate([a1[0, :, :HALF], a1[1, :, :HALF]], axis=1)

    out = pl.pallas_call(
        _stage_e_body,
        grid=grid,
        in_specs=[
            row2((B, D)),
            pl.BlockSpec((NUM_REL, B, D), lambda i: (0, i, 0)),
            pl.BlockSpec((NUM_REL, B, D), lambda i: (0, i, 0)),
            full((D, D)), full((1, D)), full((D, 2)), full((1, 2)),
        ],
        out_specs=row2((B, 2)),
        out_shape=jax.ShapeDtypeStruct((N, 2), f32),
    )(r1, aggr1, cntf, W_o1, bo1, W_o2, bo2)
    return out


# speedup vs baseline: 1.0705x; 1.0705x over previous
import functools

import jax
import jax.numpy as jnp
from jax import lax
from jax.experimental import pallas as pl
from jax.experimental.pallas import tpu as pltpu
from jax.experimental.pallas import tpu_sc as plsc

N = 10000
E = 320000
D = 128
Q = 32
NUM_REL = 2
HALF = N // 2
P = HALF + 120
LPR = 128
EROWS = 2560
EPAD = EROWS * LPR
RPT = EROWS // 16
SUP = 32
OPT = P // 16
ZPT = 2 * P // 16


def _lrelu(x):
    return jnp.where(x > 0, x, 0.01 * x)


def _stage_a_body(des, tweet, prop, wd, bd, wt, bt, wp, bp, wi, bi,
                  rw, root, rbias, h_out, r_out):
    d = _lrelu(jnp.dot(des[...], wd[...], preferred_element_type=jnp.float32) + bd[...])
    t = _lrelu(jnp.dot(tweet[...], wt[...], preferred_element_type=jnp.float32) + bt[...])
    p = _lrelu(jnp.dot(prop[...], wp[...], preferred_element_type=jnp.float32) + bp[...])
    x = jnp.concatenate((d, t, p), axis=1)
    x = _lrelu(jnp.dot(x, wi[...], preferred_element_type=jnp.float32) + bi[...])
    h_out[0] = jnp.dot(x, rw[0], preferred_element_type=jnp.float32)
    h_out[1] = jnp.dot(x, rw[1], preferred_element_type=jnp.float32)
    r_out[...] = jnp.dot(x, root[...], preferred_element_type=jnp.float32) + rbias[...]


def _stage_c_body(r0, agg, cnt, rw, root, rbias, h_out, r_out):
    c0 = jnp.maximum(cnt[0, 0][:, 0:1], 1.0)
    c1 = jnp.maximum(cnt[0, 1][:, 0:1], 1.0)
    x = r0[...] + agg[0, 0] / c0 + agg[0, 1] / c1
    h_out[0] = jnp.dot(x, rw[0], preferred_element_type=jnp.float32)
    h_out[1] = jnp.dot(x, rw[1], preferred_element_type=jnp.float32)
    r_out[...] = jnp.dot(x, root[...], preferred_element_type=jnp.float32) + rbias[...]


def _stage_e_body(r1, agg, cnt, wo1, bo1, wo2, bo2, out):
    c0 = jnp.maximum(cnt[0, 0][:, 0:1], 1.0)
    c1 = jnp.maximum(cnt[0, 1][:, 0:1], 1.0)
    x = r1[...] + agg[0, 0] / c0 + agg[0, 1] / c1
    h = _lrelu(jnp.dot(x, wo1[...], preferred_element_type=jnp.float32) + bo1[...])
    lg = jnp.dot(h, wo2[...], preferred_element_type=jnp.float32) + bo2[...]
    m = jnp.max(lg, axis=-1, keepdims=True)
    e = jnp.exp(lg - m)
    out[...] = e / jnp.sum(e, axis=-1, keepdims=True)


def _scatter_idx(dstb, typb, r, base, trash, sall):
    for l in range(8):
        sl = pl.ds(l * 16, 16)
        dv = dstb[r, sl]
        tv = typb[r, sl]
        rel = tv * P + (dv - base)
        ok = (dv >= base) & (dv < base + HALF)
        sall[r, sl] = jnp.where(ok, rel, trash)


def _conv_body(h, srcr, dstr, typr, z128,
               a_out, acc, dstb, gall, sall,
               rows_a, rows_b, sem_a, sem_b):
    c = lax.axis_index("c")
    s = lax.axis_index("s")
    lane = lax.iota(jnp.int32, 16)
    trash = HALF + (lane & 7) + P * (lane >> 3)
    base = c * HALF

    pltpu.sync_copy(z128, rows_a)
    for k in range(ZPT // 128):
        pltpu.sync_copy(rows_a, acc.at[pl.ds(s * ZPT + k * 128, 128)])
    plsc.subcore_barrier()

    def idx_step(r, _):
        for l in range(8):
            sl = pl.ds(l * 16, 16)
            sv = gall[r, sl]
            tv = sall[r, sl]
            dv = dstb[r, sl]
            rel = tv * P + (dv - base)
            ok = (dv >= base) & (dv < base + HALF)
            gall[r, sl] = tv * N + sv
            sall[r, sl] = jnp.where(ok, rel, trash)
        return _

    def wait_gather(r, rows, sem):
        pltpu.make_async_copy(h.at[gall.at[r]], rows, sem).wait()

    def pair_step(k, _):
        r0 = 2 * k
        pltpu.async_copy(h.at[gall.at[r0 + 1]], rows_b, sem_b)
        wait_gather(r0, rows_a, sem_a)
        pltpu.sync_copy(rows_a, acc.at[sall.at[r0]], add=True)

        @pl.when(k < SUP // 2 - 1)
        def _prefetch():
            pltpu.async_copy(h.at[gall.at[r0 + 2]], rows_a, sem_a)

        wait_gather(r0 + 1, rows_b, sem_b)
        pltpu.sync_copy(rows_b, acc.at[sall.at[r0 + 1]], add=True)
        return _

    for sc_i in range(RPT // SUP):
        j0 = s * RPT + sc_i * SUP
        pltpu.sync_copy(srcr.at[pl.ds(j0, SUP)], gall)
        pltpu.sync_copy(dstr.at[pl.ds(j0, SUP)], dstb)
        pltpu.sync_copy(typr.at[pl.ds(j0, SUP)], sall)
        lax.fori_loop(0, SUP, idx_step, None)
        pltpu.async_copy(h.at[gall.at[0]], rows_a, sem_a)
        lax.fori_loop(0, SUP // 2, pair_step, None)

    plsc.subcore_barrier()

    for t in range(2):
        pltpu.sync_copy(acc.at[pl.ds(t * P + s * OPT, OPT)],
                        a_out.at[c, t, pl.ds(s * OPT, OPT)])


def _count_body(dstr, typr, z128, o128,
                c_out, cacc, dstb, sall, obuf):
    c = lax.axis_index("c")
    s = lax.axis_index("s")
    lane = lax.iota(jnp.int32, 16)
    trash = HALF + (lane & 7) + P * (lane >> 3)
    base = c * HALF

    pltpu.sync_copy(z128, obuf)
    for k in range(ZPT // 128):
        pltpu.sync_copy(obuf, cacc.at[pl.ds(s * ZPT + k * 128, 128)])
    pltpu.sync_copy(o128, obuf)
    plsc.subcore_barrier()

    def row_step(r, _):
        for l in range(8):
            sl = pl.ds(l * 16, 16)
            dv = dstb[r, sl]
            tv = sall[r, sl]
            rel = tv * P + (dv - base)
            ok = (dv >= base) & (dv < base + HALF)
            sall[r, sl] = jnp.where(ok, rel, trash)
        pltpu.sync_copy(obuf, cacc.at[sall.at[r]], add=True)
        return _

    for sc_i in range(RPT // SUP):
        j0 = s * RPT + sc_i * SUP
        pltpu.sync_copy(dstr.at[pl.ds(j0, SUP)], dstb)
        pltpu.sync_copy(typr.at[pl.ds(j0, SUP)], sall)
        lax.fori_loop(0, SUP, row_step, None)

    plsc.subcore_barrier()
    for t in range(2):
        pltpu.sync_copy(cacc.at[pl.ds(t * P + s * OPT, OPT)],
                        c_out.at[c, t, pl.ds(s * OPT, OPT)])


def _sc_mesh():
    return plsc.VectorSubcoreMesh(core_axis_name="c", subcore_axis_name="s",
                                  num_cores=2, num_subcores=16)


def _make_conv(interpret=False):
    f32 = jnp.float32
    return pl.kernel(
        _conv_body,
        out_type=jax.ShapeDtypeStruct((2, 2, P, 128), f32),
        mesh=_sc_mesh(),
        scratch_types=[
            pltpu.VMEM_SHARED((2 * P, 128), f32),
            pltpu.VMEM((SUP, 128), jnp.int32),
            pltpu.VMEM((SUP, 128), jnp.int32),
            pltpu.VMEM((SUP, 128), jnp.int32),
            pltpu.VMEM((128, 128), f32),
            pltpu.VMEM((128, 128), f32),
            pltpu.SemaphoreType.DMA,
            pltpu.SemaphoreType.DMA,
        ],
        interpret=interpret,
    )


def _make_count(interpret=False):
    f32 = jnp.float32
    return pl.kernel(
        _count_body,
        out_type=jax.ShapeDtypeStruct((2, 2, P, 128), f32),
        mesh=_sc_mesh(),
        scratch_types=[
            pltpu.VMEM_SHARED((2 * P, 128), f32),
            pltpu.VMEM((SUP, 128), jnp.int32),
            pltpu.VMEM((SUP, 128), jnp.int32),
            pltpu.VMEM((128, 128), f32),
        ],
        interpret=interpret,
    )


def kernel(des, tweet, prop, edge_index, edge_type,
           W_des, b_des, W_tweet, b_tweet, W_prop, b_prop,
           W_in, b_in, rgcn_weight, rgcn_root, rgcn_bias,
           W_o1, b_o1, W_o2, b_o2):
    f32 = jnp.float32
    B = 400
    grid = (N // B,)

    full = lambda shp: pl.BlockSpec(shp, lambda i: (0,) * len(shp))
    row2 = lambda shp: pl.BlockSpec(shp, lambda i: (i, 0))

    bd = b_des.reshape(1, Q)
    bt = b_tweet.reshape(1, Q)
    bp = b_prop.reshape(1, Q)
    bi = b_in.reshape(1, D)
    rb = rgcn_bias.reshape(1, D)
    bo1 = b_o1.reshape(1, D)
    bo2 = b_o2.reshape(1, 2)

    h0, r0 = pl.pallas_call(
        _stage_a_body,
        grid=grid,
        in_specs=[
            row2((B, 768)), row2((B, 768)), row2((B, 14)),
            full((768, Q)), full((1, Q)),
            full((768, Q)), full((1, Q)),
            full((14, Q)), full((1, Q)),
            full((3 * Q, D)), full((1, D)),
            full((NUM_REL, D, D)), full((D, D)), full((1, D)),
        ],
        out_specs=[
            pl.BlockSpec((NUM_REL, B, D), lambda i: (0, i, 0)),
            row2((B, D)),
        ],
        out_shape=[
            jax.ShapeDtypeStruct((NUM_REL, N, D), f32),
            jax.ShapeDtypeStruct((N, D), f32),
        ],
    )(des, tweet, prop, W_des, bd, W_tweet, bt, W_prop, bp, W_in, bi,
      rgcn_weight, rgcn_root, rb)

    pad = EPAD - E
    src = jnp.concatenate(
        [edge_index[0], (jnp.arange(pad, dtype=jnp.int32) * 97) % N]
    ).reshape(EROWS, LPR)
    dst = jnp.concatenate(
        [edge_index[1], jnp.full((pad,), -1, jnp.int32)]).reshape(EROWS, LPR)
    typ = jnp.concatenate(
        [edge_type, jnp.zeros((pad,), jnp.int32)]).reshape(EROWS, LPR)
    z128 = jnp.zeros((128, 128), f32)
    o128 = jnp.ones((128, 128), f32)

    cnt = _make_count()(dst, typ, z128, o128)

    a0 = _make_conv()(h0.reshape(NUM_REL * N, D), src, dst, typ, z128)

    B2 = 200
    NBC = HALF // B2
    grid2 = (2, NBC)
    full2 = lambda shp: pl.BlockSpec(shp, lambda c, i: (0,) * len(shp))
    row2b = lambda shp: pl.BlockSpec(
        shp, lambda c, i: (c * NBC + i,) + (0,) * (len(shp) - 1))
    scspec = pl.BlockSpec((1, 2, B2, D), lambda c, i: (c, 0, i, 0))

    h1, r1 = pl.pallas_call(
        _stage_c_body,
        grid=grid2,
        in_specs=[
            row2b((B2, D)),
            scspec,
            scspec,
            full2((NUM_REL, D, D)), full2((D, D)), full2((1, D)),
        ],
        out_specs=[
            pl.BlockSpec((NUM_REL, B2, D), lambda c, i: (0, c * NBC + i, 0)),
            row2b((B2, D)),
        ],
        out_shape=[
            jax.ShapeDtypeStruct((NUM_REL, N, D), f32),
            jax.ShapeDtypeStruct((N, D), f32),
        ],
    )(r0, a0, cnt, rgcn_weight, rgcn_root, rb)

    a1 = _make_conv()(h1.reshape(NUM_REL * N, D), src, dst, typ, z128)

    out = pl.pallas_call(
        _stage_e_body,
        grid=grid2,
        in_specs=[
            row2b((B2, D)),
            scspec,
            scspec,
            full2((D, D)), full2((1, D)), full2((D, 2)), full2((1, 2)),
        ],
        out_specs=row2b((B2, 2)),
        out_shape=jax.ShapeDtypeStruct((N, 2), f32),
    )(r1, a1, cnt, W_o1, bo1, W_o2, bo2)
    return out


# speedup vs baseline: 1.0831x; 1.0118x over previous
import jax
import jax.numpy as jnp
from jax import lax
from jax.experimental import pallas as pl
from jax.experimental.pallas import tpu as pltpu
from jax.experimental.pallas import tpu_sc as plsc

N = 10000
E = 320000
D = 128
Q = 32
NUM_REL = 2
HALF = N // 2
P = HALF + 120
LPR = 128
EROWS = 2560
EPAD = EROWS * LPR
RPT = EROWS // 16
SUP = 32
OPT = P // 16
ZPT = 2 * P // 16


def _lrelu(x):
    return jnp.where(x > 0, x, 0.01 * x)


def _stage_a_body(des, tweet, prop, wd, bd, wt, bt, wp, bp, wi, bi,
                  rw, root, rbias, h_out, r_out):
    d = _lrelu(jnp.dot(des[...], wd[...], preferred_element_type=jnp.float32) + bd[...])
    t = _lrelu(jnp.dot(tweet[...], wt[...], preferred_element_type=jnp.float32) + bt[...])
    p = _lrelu(jnp.dot(prop[...], wp[...], preferred_element_type=jnp.float32) + bp[...])
    x = jnp.concatenate((d, t, p), axis=1)
    x = _lrelu(jnp.dot(x, wi[...], preferred_element_type=jnp.float32) + bi[...])
    h_out[0] = jnp.dot(x, rw[0], preferred_element_type=jnp.float32)
    h_out[1] = jnp.dot(x, rw[1], preferred_element_type=jnp.float32)
    r_out[...] = jnp.dot(x, root[...], preferred_element_type=jnp.float32) + rbias[...]


def _stage_c_body(r0, agg, cnt, rw, root, rbias, h_out, r_out):
    c0 = jnp.maximum(cnt[0][:, 0:1], 1.0)
    c1 = jnp.maximum(cnt[1][:, 0:1], 1.0)
    x = r0[...] + agg[0] / c0 + agg[1] / c1
    h_out[0] = jnp.dot(x, rw[0], preferred_element_type=jnp.float32)
    h_out[1] = jnp.dot(x, rw[1], preferred_element_type=jnp.float32)
    r_out[...] = jnp.dot(x, root[...], preferred_element_type=jnp.float32) + rbias[...]


def _stage_e_body(r1, agg, cnt, wo1, bo1, wo2, bo2, out):
    c0 = jnp.maximum(cnt[0][:, 0:1], 1.0)
    c1 = jnp.maximum(cnt[1][:, 0:1], 1.0)
    x = r1[...] + agg[0] / c0 + agg[1] / c1
    h = _lrelu(jnp.dot(x, wo1[...], preferred_element_type=jnp.float32) + bo1[...])
    lg = jnp.dot(h, wo2[...], preferred_element_type=jnp.float32) + bo2[...]
    m = jnp.max(lg, axis=-1, keepdims=True)
    e = jnp.exp(lg - m)
    out[...] = e / jnp.sum(e, axis=-1, keepdims=True)


def _conv_body(h, srcr, dstr, typr, z128,
               a_out, acc, dstb, gall, sall,
               rows_a, rows_b, sem_a, sem_b):
    c = lax.axis_index("c")
    s = lax.axis_index("s")
    lane = lax.iota(jnp.int32, 16)
    trash = HALF + (lane & 7) + P * (lane >> 3)
    base = c * HALF

    pltpu.sync_copy(z128, rows_a)
    for k in range(ZPT // 128):
        pltpu.sync_copy(rows_a, acc.at[pl.ds(s * ZPT + k * 128, 128)])
    plsc.subcore_barrier()

    def idx_step(r, _):
        for l in range(8):
            sl = pl.ds(l * 16, 16)
            sv = gall[r, sl]
            tv = sall[r, sl]
            dv = dstb[r, sl]
            rel = tv * P + (dv - base)
            ok = (dv >= base) & (dv < base + HALF)
            gall[r, sl] = tv * N + sv
            sall[r, sl] = jnp.where(ok, rel, trash)
        return _

    def wait_gather(r, rows, sem):
        pltpu.make_async_copy(h.at[gall.at[r]], rows, sem).wait()

    def pair_step(k, _):
        r0 = 2 * k
        pltpu.async_copy(h.at[gall.at[r0 + 1]], rows_b, sem_b)
        wait_gather(r0, rows_a, sem_a)
        pltpu.sync_copy(rows_a, acc.at[sall.at[r0]], add=True)

        @pl.when(k < SUP // 2 - 1)
        def _prefetch():
            pltpu.async_copy(h.at[gall.at[r0 + 2]], rows_a, sem_a)

        wait_gather(r0 + 1, rows_b, sem_b)
        pltpu.sync_copy(rows_b, acc.at[sall.at[r0 + 1]], add=True)
        return _

    for sc_i in range(RPT // SUP):
        j0 = s * RPT + sc_i * SUP
        pltpu.sync_copy(srcr.at[pl.ds(j0, SUP)], gall)
        pltpu.sync_copy(dstr.at[pl.ds(j0, SUP)], dstb)
        pltpu.sync_copy(typr.at[pl.ds(j0, SUP)], sall)
        lax.fori_loop(0, SUP, idx_step, None)
        pltpu.async_copy(h.at[gall.at[0]], rows_a, sem_a)
        lax.fori_loop(0, SUP // 2, pair_step, None)

    plsc.subcore_barrier()

    for t in range(2):
        pltpu.sync_copy(acc.at[pl.ds(t * P + s * OPT, OPT)],
                        a_out.at[c, t, pl.ds(s * OPT, OPT)])


def _count_body(dstr, typr, z128, o128,
                c_out, cacc, dstb, sall, obuf):
    c = lax.axis_index("c")
    s = lax.axis_index("s")
    lane = lax.iota(jnp.int32, 16)
    trash = HALF + (lane & 7) + P * (lane >> 3)
    base = c * HALF

    pltpu.sync_copy(z128, obuf)
    for k in range(ZPT // 128):
        pltpu.sync_copy(obuf, cacc.at[pl.ds(s * ZPT + k * 128, 128)])
    pltpu.sync_copy(o128, obuf)
    plsc.subcore_barrier()

    def row_step(r, _):
        for l in range(8):
            sl = pl.ds(l * 16, 16)
            dv = dstb[r, sl]
            tv = sall[r, sl]
            rel = tv * P + (dv - base)
            ok = (dv >= base) & (dv < base + HALF)
            sall[r, sl] = jnp.where(ok, rel, trash)
        pltpu.sync_copy(obuf, cacc.at[sall.at[r]], add=True)
        return _

    for sc_i in range(RPT // SUP):
        j0 = s * RPT + sc_i * SUP
        pltpu.sync_copy(dstr.at[pl.ds(j0, SUP)], dstb)
        pltpu.sync_copy(typr.at[pl.ds(j0, SUP)], sall)
        lax.fori_loop(0, SUP, row_step, None)

    plsc.subcore_barrier()
    for t in range(2):
        pltpu.sync_copy(cacc.at[pl.ds(t * P + s * OPT, OPT)],
                        c_out.at[c, t, pl.ds(s * OPT, OPT)])


def _sc_mesh():
    return plsc.VectorSubcoreMesh(core_axis_name="c", subcore_axis_name="s",
                                  num_cores=2, num_subcores=16)


def _make_conv(interpret=False):
    f32 = jnp.float32
    return pl.kernel(
        _conv_body,
        out_type=jax.ShapeDtypeStruct((2, 2, P, 128), f32),
        mesh=_sc_mesh(),
        scratch_types=[
            pltpu.VMEM_SHARED((2 * P, 128), f32),
            pltpu.VMEM((SUP, 128), jnp.int32),
            pltpu.VMEM((SUP, 128), jnp.int32),
            pltpu.VMEM((SUP, 128), jnp.int32),
            pltpu.VMEM((128, 128), f32),
            pltpu.VMEM((128, 128), f32),
            pltpu.SemaphoreType.DMA,
            pltpu.SemaphoreType.DMA,
        ],
        interpret=interpret,
    )


def _make_count(interpret=False):
    f32 = jnp.float32
    return pl.kernel(
        _count_body,
        out_type=jax.ShapeDtypeStruct((2, 2, P, 128), f32),
        mesh=_sc_mesh(),
        scratch_types=[
            pltpu.VMEM_SHARED((2 * P, 128), f32),
            pltpu.VMEM((SUP, 128), jnp.int32),
            pltpu.VMEM((SUP, 128), jnp.int32),
            pltpu.VMEM((128, 128), f32),
        ],
        interpret=interpret,
    )


def kernel(des, tweet, prop, edge_index, edge_type,
           W_des, b_des, W_tweet, b_tweet, W_prop, b_prop,
           W_in, b_in, rgcn_weight, rgcn_root, rgcn_bias,
           W_o1, b_o1, W_o2, b_o2):
    f32 = jnp.float32
    B = 400
    grid = (N // B,)

    full = lambda shp: pl.BlockSpec(shp, lambda i: (0,) * len(shp))
    row2 = lambda shp: pl.BlockSpec(shp, lambda i: (i, 0))

    bd = b_des.reshape(1, Q)
    bt = b_tweet.reshape(1, Q)
    bp = b_prop.reshape(1, Q)
    bi = b_in.reshape(1, D)
    rb = rgcn_bias.reshape(1, D)
    bo1 = b_o1.reshape(1, D)
    bo2 = b_o2.reshape(1, 2)

    h0, r0 = pl.pallas_call(
        _stage_a_body,
        grid=grid,
        in_specs=[
            row2((B, 768)), row2((B, 768)), row2((B, 14)),
            full((768, Q)), full((1, Q)),
            full((768, Q)), full((1, Q)),
            full((14, Q)), full((1, Q)),
            full((3 * Q, D)), full((1, D)),
            full((NUM_REL, D, D)), full((D, D)), full((1, D)),
        ],
        out_specs=[
            pl.BlockSpec((NUM_REL, B, D), lambda i: (0, i, 0)),
            row2((B, D)),
        ],
        out_shape=[
            jax.ShapeDtypeStruct((NUM_REL, N, D), f32),
            jax.ShapeDtypeStruct((N, D), f32),
        ],
    )(des, tweet, prop, W_des, bd, W_tweet, bt, W_prop, bp, W_in, bi,
      rgcn_weight, rgcn_root, rb)

    pad = EPAD - E
    src = jnp.concatenate(
        [edge_index[0], (jnp.arange(pad, dtype=jnp.int32) * 97) % N]
    ).reshape(EROWS, LPR)
    dst = jnp.concatenate(
        [edge_index[1], jnp.full((pad,), -1, jnp.int32)]).reshape(EROWS, LPR)
    typ = jnp.concatenate(
        [edge_type, jnp.zeros((pad,), jnp.int32)]).reshape(EROWS, LPR)
    z128 = jnp.zeros((128, 128), f32)
    o128 = jnp.ones((128, 128), f32)

    cnt = _make_count()(dst, typ, z128, o128)
    cntf = jnp.concatenate([cnt[0, :, :HALF], cnt[1, :, :HALF]], axis=1)

    a0 = _make_conv()(h0.reshape(NUM_REL * N, D), src, dst, typ, z128)
    aggr0 = jnp.concatenate([a0[0, :, :HALF], a0[1, :, :HALF]], axis=1)

    h1, r1 = pl.pallas_call(
        _stage_c_body,
        grid=grid,
        in_specs=[
            row2((B, D)),
            pl.BlockSpec((NUM_REL, B, D), lambda i: (0, i, 0)),
            pl.BlockSpec((NUM_REL, B, D), lambda i: (0, i, 0)),
            full((NUM_REL, D, D)), full((D, D)), full((1, D)),
        ],
        out_specs=[
            pl.BlockSpec((NUM_REL, B, D), lambda i: (0, i, 0)),
            row2((B, D)),
        ],
        out_shape=[
            jax.ShapeDtypeStruct((NUM_REL, N, D), f32),
            jax.ShapeDtypeStruct((N, D), f32),
        ],
    )(r0, aggr0, cntf, rgcn_weight, rgcn_root, rb)

    a1 = _make_conv()(h1.reshape(NUM_REL * N, D), src, dst, typ, z128)
    aggr1 = jnp.concatenate([a1[0, :, :HALF], a1[1, :, :HALF]], axis=1)

    out = pl.pallas_call(
        _stage_e_body,
        grid=grid,
        in_specs=[
            row2((B, D)),
            pl.BlockSpec((NUM_REL, B, D), lambda i: (0, i, 0)),
            pl.BlockSpec((NUM_REL, B, D), lambda i: (0, i, 0)),
            full((D, D)), full((1, D)), full((D, 2)), full((1, 2)),
        ],
        out_specs=row2((B, 2)),
        out_shape=jax.ShapeDtypeStruct((N, 2), f32),
    )(r1, aggr1, cntf, W_o1, bo1, W_o2, bo2)
    return out
